# baseline (speedup 1.0000x reference)
import jax
import jax.numpy as jnp
from jax import lax
from jax.experimental import pallas as pl
from jax.experimental.pallas import tpu as pltpu
from jax.experimental.pallas import tpu_sc as plsc

NC, NS, L = 2, 16, 16
NW = NC * NS
B = 16384
BPW = B // NW
CHUNK = 128
NCHUNK = BPW // CHUNK
GPC = CHUNK // L
D_DRV, D_WEEK, D_TIME = 16, 3, 8
D_OUT = D_DRV + D_WEEK + D_TIME + 1


def _body(drv_hbm, wk_hbm, tm_hbm, dist_hbm, Wd_hbm, Ww_hbm, Wt_hbm,
          out_hbm,
          didx_v, tidx_v, widx_v, dist_v, drv_v, tm_v, ww_v, col_v,
          sem_in, sem_g, sem_out):
  wid = lax.axis_index("s") * NC + lax.axis_index("c")
  base = wid * BPW

  pltpu.async_copy(drv_hbm.at[pl.ds(base, BPW)], didx_v, sem_in)
  pltpu.async_copy(tm_hbm.at[pl.ds(base, BPW)], tidx_v, sem_in)
  pltpu.async_copy(wk_hbm.at[pl.ds(base, BPW)], widx_v, sem_in)
  pltpu.async_copy(dist_hbm.at[pl.ds(base, BPW)], dist_v, sem_in)
  pltpu.async_copy(Ww_hbm, ww_v, sem_in)
  pltpu.make_async_copy(drv_hbm.at[pl.ds(base, BPW)], didx_v, sem_in).wait()
  pltpu.make_async_copy(tm_hbm.at[pl.ds(base, BPW)], tidx_v, sem_in).wait()
  pltpu.make_async_copy(wk_hbm.at[pl.ds(base, BPW)], widx_v, sem_in).wait()
  pltpu.make_async_copy(dist_hbm.at[pl.ds(base, BPW)], dist_v, sem_in).wait()
  pltpu.make_async_copy(Ww_hbm, ww_v, sem_in).wait()

  for c in range(NCHUNK):
    pltpu.async_copy(Wd_hbm.at[didx_v.at[pl.ds(c * CHUNK, CHUNK)]],
                     drv_v.at[c], sem_g)
    pltpu.async_copy(Wt_hbm.at[tidx_v.at[pl.ds(c * CHUNK, CHUNK)]],
                     tm_v.at[c], sem_g)

  lane = lax.iota(jnp.int32, L)

  for c in range(NCHUNK):
    pltpu.make_async_copy(Wd_hbm.at[didx_v.at[pl.ds(c * CHUNK, CHUNK)]],
                          drv_v.at[c], sem_g).wait()
    pltpu.make_async_copy(Wt_hbm.at[tidx_v.at[pl.ds(c * CHUNK, CHUNK)]],
                          tm_v.at[c], sem_g).wait()
    chv = jnp.full((L,), c, jnp.int32)

    def group(g, _):
      rbase = c * CHUNK + g * L
      rloc = g * L
      rv = rloc + lane

      for cc in range(D_DRV):
        v = plsc.load_gather(drv_v, [chv, rv, jnp.full((L,), cc, jnp.int32)])
        col_v[cc, pl.ds(rbase, L)] = v

      wvec = widx_v[pl.ds(rbase, L)]
      for cc in range(D_WEEK):
        v = plsc.load_gather(ww_v, [wvec, jnp.full((L,), cc, jnp.int32)])
        col_v[D_DRV + cc, pl.ds(rbase, L)] = v

      for cc in range(D_TIME):
        v = plsc.load_gather(tm_v, [chv, rv, jnp.full((L,), cc, jnp.int32)])
        col_v[D_DRV + D_WEEK + cc, pl.ds(rbase, L)] = v

      col_v[D_OUT - 1, pl.ds(rbase, L)] = dist_v[pl.ds(rbase, L)]
      return 0

    lax.fori_loop(0, GPC, group, 0)

  for cc in range(D_OUT):
    pltpu.async_copy(col_v.at[cc], out_hbm.at[pl.ds(cc * B + base, BPW)],
                     sem_out)
  for cc in range(D_OUT):
    pltpu.make_async_copy(col_v.at[cc], out_hbm.at[pl.ds(cc * B + base, BPW)],
                          sem_out).wait()


def _tc_flatten_rows(table_t):
  D, V = table_t.shape

  def body(x_ref, o_ref):
    o_ref[...] = jnp.reshape(x_ref[...].T, (V * D,))

  return pl.pallas_call(
      body, out_shape=jax.ShapeDtypeStruct((V * D,), jnp.float32))(table_t)


@jax.jit
def _run(driverID, weekID, timeID, dist, W_driver, W_week, W_time):
  mesh = plsc.VectorSubcoreMesh(core_axis_name="c", subcore_axis_name="s")
  out = pl.kernel(
      _body,
      out_type=jax.ShapeDtypeStruct((D_OUT * B,), jnp.float32),
      mesh=mesh,
      compiler_params=pltpu.CompilerParams(needs_layout_passes=False,
                                           use_tc_tiling_on_sc=False),
      scratch_types=[
          pltpu.VMEM((BPW,), jnp.int32),
          pltpu.VMEM((BPW,), jnp.int32),
          pltpu.VMEM((BPW,), jnp.int32),
          pltpu.VMEM((BPW,), jnp.float32),
          pltpu.VMEM((NCHUNK, CHUNK, D_DRV), jnp.float32),
          pltpu.VMEM((NCHUNK, CHUNK, D_TIME), jnp.float32),
          pltpu.VMEM((7, D_WEEK), jnp.float32),
          pltpu.VMEM((D_OUT, BPW), jnp.float32),
          pltpu.SemaphoreType.DMA,
          pltpu.SemaphoreType.DMA,
          pltpu.SemaphoreType.DMA,
      ],
  )(driverID, weekID, timeID, dist, W_driver, W_week, W_time)
  return out.reshape(D_OUT, B).T


def kernel(driverID, weekID, timeID, dist, W_driver, W_week, W_time):
  return _run(driverID.astype(jnp.int32), weekID.astype(jnp.int32),
              timeID.astype(jnp.int32), dist.astype(jnp.float32),
              W_driver, W_week, W_time)


# speedup vs baseline: 1.4390x; 1.4390x over previous
import jax
import jax.numpy as jnp
from jax import lax
from jax.experimental import pallas as pl
from jax.experimental.pallas import tpu as pltpu
from jax.experimental.pallas import tpu_sc as plsc

NC, NS, L = 2, 16, 16
B = 16384
V_DRV, V_WEEK, V_TIME = 24000, 7, 1440
D_DRV, D_WEEK, D_TIME = 16, 3, 8
D_OUT = D_DRV + D_WEEK + D_TIME + 1
UNROLL = 8
GROUPS = B // L


def _body(drv_hbm, wk_hbm, tm_hbm, dist_hbm, Wd_hbm, Ww_hbm, Wt_hbm,
          out_hbm, idx_v, tab_v, col_v, sem):
  wid = lax.axis_index("s") * NC + lax.axis_index("c")

  @pl.when(wid < D_DRV)
  def _():
    pltpu.async_copy(Wd_hbm.at[wid], tab_v.at[pl.ds(0, V_DRV)], sem)
    pltpu.async_copy(drv_hbm, idx_v, sem)
    pltpu.make_async_copy(Wd_hbm.at[wid], tab_v.at[pl.ds(0, V_DRV)], sem).wait()
    pltpu.make_async_copy(drv_hbm, idx_v, sem).wait()

  @pl.when((wid >= D_DRV) & (wid < D_DRV + D_WEEK))
  def _():
    pltpu.async_copy(Ww_hbm.at[wid - D_DRV], tab_v.at[pl.ds(0, V_WEEK)], sem)
    pltpu.async_copy(wk_hbm, idx_v, sem)
    pltpu.make_async_copy(Ww_hbm.at[wid - D_DRV], tab_v.at[pl.ds(0, V_WEEK)],
                          sem).wait()
    pltpu.make_async_copy(wk_hbm, idx_v, sem).wait()

  @pl.when((wid >= D_DRV + D_WEEK) & (wid < D_OUT - 1))
  def _():
    t = wid - (D_DRV + D_WEEK)
    pltpu.async_copy(Wt_hbm.at[t], tab_v.at[pl.ds(0, V_TIME)], sem)
    pltpu.async_copy(tm_hbm, idx_v, sem)
    pltpu.make_async_copy(Wt_hbm.at[t], tab_v.at[pl.ds(0, V_TIME)], sem).wait()
    pltpu.make_async_copy(tm_hbm, idx_v, sem).wait()

  @pl.when(wid < D_OUT - 1)
  def _():
    def step(i, _):
      for u in range(UNROLL):
        off = (i * UNROLL + u) * L
        iv = idx_v[pl.ds(off, L)]
        col_v[pl.ds(off, L)] = plsc.load_gather(tab_v, [iv])
      return 0

    lax.fori_loop(0, GROUPS // UNROLL, step, 0)
    pltpu.sync_copy(col_v, out_hbm.at[pl.ds(wid * B, B)])

  @pl.when(wid == D_OUT - 1)
  def _():
    pltpu.sync_copy(dist_hbm, col_v)
    pltpu.sync_copy(col_v, out_hbm.at[pl.ds(wid * B, B)])


@jax.jit
def _run(driverID, weekID, timeID, dist, W_driver, W_week, W_time):
  mesh = plsc.VectorSubcoreMesh(core_axis_name="c", subcore_axis_name="s")
  out = pl.kernel(
      _body,
      out_type=jax.ShapeDtypeStruct((D_OUT * B,), jnp.float32),
      mesh=mesh,
      compiler_params=pltpu.CompilerParams(needs_layout_passes=False,
                                           use_tc_tiling_on_sc=False),
      scratch_types=[
          pltpu.VMEM((B,), jnp.int32),
          pltpu.VMEM((V_DRV,), jnp.float32),
          pltpu.VMEM((B,), jnp.float32),
          pltpu.SemaphoreType.DMA,
      ],
  )(driverID, weekID, timeID, dist,
    W_driver.T, W_week.T, W_time.T)
  return out.reshape(D_OUT, B).T


def kernel(driverID, weekID, timeID, dist, W_driver, W_week, W_time):
  return _run(driverID.astype(jnp.int32), weekID.astype(jnp.int32),
              timeID.astype(jnp.int32), dist.astype(jnp.float32),
              W_driver, W_week, W_time)
